# initial kernel scaffold (unmeasured)
import jax
import jax.numpy as jnp
from jax import lax
from jax.experimental import pallas as pl
from jax.experimental.pallas import tpu as pltpu


def kernel(
    x,
):
    def body(*refs):
        pass

    out_shape = jax.ShapeDtypeStruct(..., jnp.float32)
    return pl.pallas_call(body, out_shape=out_shape)(...)


# baseline (device time: 50586 ns/iter reference)
import jax
import jax.numpy as jnp
from jax import lax
from jax.experimental import pallas as pl
from jax.experimental.pallas import tpu as pltpu

K = 16
PAD = 128


def _topk_desc(v, k):
    rows, n = v.shape
    iota = lax.broadcasted_iota(jnp.int32, (rows, n), 1)
    outs = []
    for _ in range(k):
        m = jnp.max(v, axis=1, keepdims=True)
        sel = jnp.where(v == m, iota, n)
        fi = jnp.min(sel, axis=1, keepdims=True)
        v = jnp.where(iota == fi, -jnp.inf, v)
        outs.append(m)
    return jnp.concatenate(outs, axis=1)


def kernel(x):
    rows, _ = x.shape

    def body(x_ref, out_ref, topk_ref, comm_ref, send_sem, recv_sem):
        mx = lax.axis_index("x")
        my = lax.axis_index("y")
        mz = lax.axis_index("z")
        partner = (mx, my, 1 - mz)

        barrier = pltpu.get_barrier_semaphore()
        pl.semaphore_signal(
            barrier, inc=1, device_id=partner,
            device_id_type=pl.DeviceIdType.MESH,
        )
        pl.semaphore_wait(barrier, 1)

        tk = _topk_desc(x_ref[:, :], K)
        topk_ref[:, :] = jnp.concatenate(
            [tk, jnp.full((rows, PAD - K), -jnp.inf, jnp.float32)], axis=1
        )

        rdma = pltpu.make_async_remote_copy(
            src_ref=topk_ref,
            dst_ref=comm_ref,
            send_sem=send_sem,
            recv_sem=recv_sem,
            device_id=partner,
            device_id_type=pl.DeviceIdType.MESH,
        )
        rdma.start()
        rdma.wait()

        merged = jnp.concatenate([tk, comm_ref[:, :K]], axis=1)
        out_ref[:, :] = _topk_desc(merged, K)

    return pl.pallas_call(
        body,
        out_shape=jax.ShapeDtypeStruct((rows, K), jnp.float32),
        in_specs=[pl.BlockSpec(memory_space=pltpu.VMEM)],
        out_specs=pl.BlockSpec(memory_space=pltpu.VMEM),
        scratch_shapes=[
            pltpu.VMEM((rows, PAD), jnp.float32),
            pltpu.VMEM((rows, PAD), jnp.float32),
            pltpu.SemaphoreType.DMA,
            pltpu.SemaphoreType.DMA,
        ],
        compiler_params=pltpu.CompilerParams(collective_id=0),
    )(x)


# device time: 20326 ns/iter; 2.4887x vs baseline; 2.4887x over previous
import jax
import jax.numpy as jnp
from jax import lax
from jax.experimental import pallas as pl
from jax.experimental.pallas import tpu as pltpu

K = 16
NXY = 4


def _topk_desc(v, k):
    outs = []
    for _ in range(k):
        m = jnp.max(v, axis=1, keepdims=True)
        outs.append(m)
        v = jnp.where(v == m, -jnp.inf, v)
    return jnp.concatenate(outs, axis=1)


def kernel(x):
    rows, _ = x.shape
    rblk = rows // NXY

    def body(x_ref, out_ref, zbuf_ref, send_sems, recv_sems):
        mx = lax.axis_index("x")
        my = lax.axis_index("y")
        mz = lax.axis_index("z")
        z_partner = (mx, my, 1 - mz)
        y_partner = (mx, 1 - my, mz)
        x_partner = (1 - mx, my, mz)

        barrier = pltpu.get_barrier_semaphore()
        for nbr in (z_partner, y_partner, x_partner):
            pl.semaphore_signal(
                barrier, inc=1, device_id=nbr,
                device_id_type=pl.DeviceIdType.MESH,
            )
        pl.semaphore_wait(barrier, 3)

        rb = 2 * mx + my
        r0 = rb * rblk

        tk = _topk_desc(x_ref[pl.ds(r0, rblk), :], K)
        zbuf_ref[0, :, :] = tk

        zx = pltpu.make_async_remote_copy(
            src_ref=zbuf_ref.at[0],
            dst_ref=zbuf_ref.at[1],
            send_sem=send_sems.at[0],
            recv_sem=recv_sems.at[0],
            device_id=z_partner,
            device_id_type=pl.DeviceIdType.MESH,
        )
        zx.start()
        zx.wait()

        merged = jnp.concatenate([tk, zbuf_ref[1, :, :]], axis=1)
        out_ref[pl.ds(r0, rblk), :] = _topk_desc(merged, K)

        yx = pltpu.make_async_remote_copy(
            src_ref=out_ref.at[pl.ds(r0, rblk), :],
            dst_ref=out_ref.at[pl.ds(r0, rblk), :],
            send_sem=send_sems.at[1],
            recv_sem=recv_sems.at[1],
            device_id=y_partner,
            device_id_type=pl.DeviceIdType.MESH,
        )
        yx.start()
        yx.wait()

        s0 = 2 * mx * rblk
        xx = pltpu.make_async_remote_copy(
            src_ref=out_ref.at[pl.ds(s0, 2 * rblk), :],
            dst_ref=out_ref.at[pl.ds(s0, 2 * rblk), :],
            send_sem=send_sems.at[2],
            recv_sem=recv_sems.at[2],
            device_id=x_partner,
            device_id_type=pl.DeviceIdType.MESH,
        )
        xx.start()
        xx.wait()

    return pl.pallas_call(
        body,
        out_shape=jax.ShapeDtypeStruct((rows, K), jnp.float32),
        in_specs=[pl.BlockSpec(memory_space=pltpu.VMEM)],
        out_specs=pl.BlockSpec(memory_space=pltpu.VMEM),
        scratch_shapes=[
            pltpu.VMEM((2, rblk, K), jnp.float32),
            pltpu.SemaphoreType.DMA((3,)),
            pltpu.SemaphoreType.DMA((3,)),
        ],
        compiler_params=pltpu.CompilerParams(collective_id=0),
    )(x)


# device time: 18355 ns/iter; 2.7560x vs baseline; 1.1074x over previous
import jax
import jax.numpy as jnp
from jax import lax
from jax.experimental import pallas as pl
from jax.experimental.pallas import tpu as pltpu

K = 16
NXY = 4


def _topk_desc(v, k):
    outs = []
    for _ in range(k):
        m = jnp.max(v, axis=1, keepdims=True)
        outs.append(m)
        v = jnp.where(v == m, -jnp.inf, v)
    return jnp.concatenate(outs, axis=1)


def kernel(x):
    rows, ncols = x.shape
    rblk = rows // NXY

    def body(x_hbm, out_ref, xblk, cand, copy_sem, send_sems, recv_sems):
        mx = lax.axis_index("x")
        my = lax.axis_index("y")
        mz = lax.axis_index("z")
        rb = 2 * mx + my
        r0 = rb * rblk

        cp = pltpu.make_async_copy(
            x_hbm.at[pl.ds(r0, rblk), :], xblk, copy_sem
        )
        cp.start()

        barrier = pltpu.get_barrier_semaphore()
        for tx in (0, 1):
            for ty in (0, 1):
                for tz in (0, 1):
                    pl.semaphore_signal(
                        barrier, inc=1, device_id=(tx, ty, tz),
                        device_id_type=pl.DeviceIdType.MESH,
                    )
        pl.semaphore_wait(barrier, 8)

        cp.wait()

        tk = _topk_desc(xblk[:, :], K)
        cand[mz, pl.ds(r0, rblk), :] = tk

        def for_each_peer(fn):
            for tx in (0, 1):
                for ty in (0, 1):
                    for tz in (0, 1):
                        not_self = jnp.logical_not(
                            (tx == mx) & (ty == my) & (tz == mz)
                        )
                        fn((tx, ty, tz), (tz, 2 * tx + ty), not_self)

        def peer_rdma(target, sem_idx):
            return pltpu.make_async_remote_copy(
                src_ref=cand.at[mz, pl.ds(r0, rblk), :],
                dst_ref=cand.at[sem_idx[0], pl.ds(sem_idx[1] * rblk, rblk), :],
                send_sem=send_sems.at[sem_idx],
                recv_sem=recv_sems.at[sem_idx],
                device_id=target,
                device_id_type=pl.DeviceIdType.MESH,
            )

        def start_send(target, tslot, not_self):
            @pl.when(not_self)
            def _():
                pltpu.make_async_remote_copy(
                    src_ref=cand.at[mz, pl.ds(r0, rblk), :],
                    dst_ref=cand.at[mz, pl.ds(r0, rblk), :],
                    send_sem=send_sems.at[tslot],
                    recv_sem=recv_sems.at[mz, rb],
                    device_id=target,
                    device_id_type=pl.DeviceIdType.MESH,
                ).start()

        for_each_peer(start_send)

        def wait_recv(target, sslot, not_self):
            @pl.when(not_self)
            def _():
                peer_rdma(target, sslot).wait_recv()

        for_each_peer(wait_recv)

        merged = jnp.concatenate([cand[0], cand[1]], axis=1)
        out_ref[:, :] = _topk_desc(merged, K)

        def wait_send(target, tslot, not_self):
            @pl.when(not_self)
            def _():
                peer_rdma(target, tslot).wait_send()

        for_each_peer(wait_send)

    return pl.pallas_call(
        body,
        out_shape=jax.ShapeDtypeStruct((rows, K), jnp.float32),
        in_specs=[pl.BlockSpec(memory_space=pl.ANY)],
        out_specs=pl.BlockSpec(memory_space=pltpu.VMEM),
        scratch_shapes=[
            pltpu.VMEM((rblk, ncols), jnp.float32),
            pltpu.VMEM((2, rows, K), jnp.float32),
            pltpu.SemaphoreType.DMA,
            pltpu.SemaphoreType.DMA((2, NXY)),
            pltpu.SemaphoreType.DMA((2, NXY)),
        ],
        compiler_params=pltpu.CompilerParams(collective_id=0),
    )(x)


# device time: 16310 ns/iter; 3.1015x vs baseline; 1.1254x over previous
import jax
import jax.numpy as jnp
from jax import lax
from jax.experimental import pallas as pl
from jax.experimental.pallas import tpu as pltpu

K = 16
NXY = 4


def _topk_desc(v, k):
    outs = []
    for _ in range(k):
        m = jnp.max(v, axis=1, keepdims=True)
        outs.append(m)
        v = jnp.where(v == m, -jnp.inf, v)
    return jnp.concatenate(outs, axis=1)


def _local_topk(v, k):
    _, n = v.shape
    w = 128
    m1 = v[:, :w]
    m2 = jnp.full_like(m1, -jnp.inf)
    for c in range(1, n // w):
        xc = v[:, c * w:(c + 1) * w]
        m2 = jnp.maximum(m2, jnp.minimum(m1, xc))
        m1 = jnp.maximum(m1, xc)
    return _topk_desc(jnp.concatenate([m1, m2], axis=1), k)


def kernel(x):
    rows, ncols = x.shape
    rblk = rows // NXY

    def body(x_hbm, out_ref, xblk, cand, copy_sem, send_sems, recv_sems):
        mx = lax.axis_index("x")
        my = lax.axis_index("y")
        mz = lax.axis_index("z")
        rb = 2 * mx + my
        r0 = rb * rblk

        cp = pltpu.make_async_copy(
            x_hbm.at[pl.ds(r0, rblk), :], xblk, copy_sem
        )
        cp.start()

        barrier = pltpu.get_barrier_semaphore()
        for tx in (0, 1):
            for ty in (0, 1):
                for tz in (0, 1):
                    pl.semaphore_signal(
                        barrier, inc=1, device_id=(tx, ty, tz),
                        device_id_type=pl.DeviceIdType.MESH,
                    )
        pl.semaphore_wait(barrier, 8)

        cp.wait()

        tk = _local_topk(xblk[:, :], K)
        cand[mz, pl.ds(r0, rblk), :] = tk

        def for_each_peer(fn):
            for tx in (0, 1):
                for ty in (0, 1):
                    for tz in (0, 1):
                        not_self = jnp.logical_not(
                            (tx == mx) & (ty == my) & (tz == mz)
                        )
                        fn((tx, ty, tz), (tz, 2 * tx + ty), not_self)

        def peer_rdma(target, sem_idx):
            return pltpu.make_async_remote_copy(
                src_ref=cand.at[mz, pl.ds(r0, rblk), :],
                dst_ref=cand.at[sem_idx[0], pl.ds(sem_idx[1] * rblk, rblk), :],
                send_sem=send_sems.at[sem_idx],
                recv_sem=recv_sems.at[sem_idx],
                device_id=target,
                device_id_type=pl.DeviceIdType.MESH,
            )

        def start_send(target, tslot, not_self):
            @pl.when(not_self)
            def _():
                pltpu.make_async_remote_copy(
                    src_ref=cand.at[mz, pl.ds(r0, rblk), :],
                    dst_ref=cand.at[mz, pl.ds(r0, rblk), :],
                    send_sem=send_sems.at[tslot],
                    recv_sem=recv_sems.at[mz, rb],
                    device_id=target,
                    device_id_type=pl.DeviceIdType.MESH,
                ).start()

        for_each_peer(start_send)

        def wait_recv(target, sslot, not_self):
            @pl.when(not_self)
            def _():
                peer_rdma(target, sslot).wait_recv()

        for_each_peer(wait_recv)

        merged = jnp.concatenate([cand[0], cand[1]], axis=1)
        out_ref[:, :] = _topk_desc(merged, K)

        def wait_send(target, tslot, not_self):
            @pl.when(not_self)
            def _():
                peer_rdma(target, tslot).wait_send()

        for_each_peer(wait_send)

    return pl.pallas_call(
        body,
        out_shape=jax.ShapeDtypeStruct((rows, K), jnp.float32),
        in_specs=[pl.BlockSpec(memory_space=pl.ANY)],
        out_specs=pl.BlockSpec(memory_space=pltpu.VMEM),
        scratch_shapes=[
            pltpu.VMEM((rblk, ncols), jnp.float32),
            pltpu.VMEM((2, rows, K), jnp.float32),
            pltpu.SemaphoreType.DMA,
            pltpu.SemaphoreType.DMA((2, NXY)),
            pltpu.SemaphoreType.DMA((2, NXY)),
        ],
        compiler_params=pltpu.CompilerParams(collective_id=0),
    )(x)


# device time: 14915 ns/iter; 3.3916x vs baseline; 1.0935x over previous
import os

import jax
import jax.numpy as jnp
from jax import lax
from jax.experimental import pallas as pl
from jax.experimental.pallas import tpu as pltpu

K = 16
NXY = 4
_ABLATE = os.environ.get("KERNEL_ABLATE", "")


def _topk_desc(v, k):
    outs = []
    for _ in range(k):
        m = jnp.max(v, axis=1, keepdims=True)
        outs.append(m)
        v = jnp.where(v == m, -jnp.inf, v)
    return jnp.concatenate(outs, axis=1)


def _local_topk(v, k):
    _, n = v.shape
    w = 128
    m1 = v[:, :w]
    m2 = jnp.full_like(m1, -jnp.inf)
    for c in range(1, n // w):
        xc = v[:, c * w:(c + 1) * w]
        m2 = jnp.maximum(m2, jnp.minimum(m1, xc))
        m1 = jnp.maximum(m1, xc)
    return _topk_desc(jnp.concatenate([m1, m2], axis=1), k)


def kernel(x):
    rows, ncols = x.shape
    rblk = rows // NXY

    def body(x_hbm, out_ref, xblk, cand, copy_sem, send_sems, recv_sems):
        mx = lax.axis_index("x")
        my = lax.axis_index("y")
        mz = lax.axis_index("z")
        rb = 2 * mx + my
        r0 = rb * rblk

        cp = pltpu.make_async_copy(
            x_hbm.at[pl.ds(r0, rblk), :], xblk, copy_sem
        )
        cp.start()

        if _ABLATE not in ("local", "minimal"):
            barrier = pltpu.get_barrier_semaphore()
            for tx in (0, 1):
                for ty in (0, 1):
                    for tz in (0, 1):
                        pl.semaphore_signal(
                            barrier, inc=1, device_id=(tx, ty, tz),
                            device_id_type=pl.DeviceIdType.MESH,
                        )

        cp.wait()

        if _ABLATE == "minimal":
            out_ref[pl.ds(r0, rblk), :] = xblk[:, :K]
            return
        if _ABLATE == "noextract":
            tk = xblk[:, :K]
        elif _ABLATE == "nostage2":
            tk = jnp.maximum(xblk[:, :K], xblk[:, K:2 * K])
            for c in range(2, 32):
                tk = jnp.maximum(tk, xblk[:, c * K:(c + 1) * K])
        else:
            tk = _local_topk(xblk[:, :], K)
        cand[mz, pl.ds(r0, rblk), :] = tk

        def for_each_peer(fn):
            for tx in (0, 1):
                for ty in (0, 1):
                    for tz in (0, 1):
                        not_self = jnp.logical_not(
                            (tx == mx) & (ty == my) & (tz == mz)
                        )
                        fn((tx, ty, tz), (tz, 2 * tx + ty), not_self)

        def peer_rdma(target, sem_idx):
            return pltpu.make_async_remote_copy(
                src_ref=cand.at[mz, pl.ds(r0, rblk), :],
                dst_ref=cand.at[sem_idx[0], pl.ds(sem_idx[1] * rblk, rblk), :],
                send_sem=send_sems.at[sem_idx],
                recv_sem=recv_sems.at[sem_idx],
                device_id=target,
                device_id_type=pl.DeviceIdType.MESH,
            )

        def start_send(target, tslot, not_self):
            @pl.when(not_self)
            def _():
                pltpu.make_async_remote_copy(
                    src_ref=cand.at[mz, pl.ds(r0, rblk), :],
                    dst_ref=cand.at[mz, pl.ds(r0, rblk), :],
                    send_sem=send_sems.at[tslot],
                    recv_sem=recv_sems.at[mz, rb],
                    device_id=target,
                    device_id_type=pl.DeviceIdType.MESH,
                ).start()

        if _ABLATE not in ("nobcast", "local"):
            pl.semaphore_wait(barrier, 8)
            for_each_peer(start_send)

            def wait_recv(target, sslot, not_self):
                @pl.when(not_self)
                def _():
                    peer_rdma(target, sslot).wait_recv()

            for_each_peer(wait_recv)

        if _ABLATE == "nomerge":
            out_ref[:, :] = cand[0]
        else:
            merged = jnp.concatenate([cand[0], cand[1]], axis=1)
            out_ref[:, :] = _topk_desc(merged, K)

        if _ABLATE not in ("nobcast", "local"):
            def wait_send(target, tslot, not_self):
                @pl.when(not_self)
                def _():
                    peer_rdma(target, tslot).wait_send()

            for_each_peer(wait_send)

    return pl.pallas_call(
        body,
        out_shape=jax.ShapeDtypeStruct((rows, K), jnp.float32),
        in_specs=[pl.BlockSpec(memory_space=pl.ANY)],
        out_specs=pl.BlockSpec(memory_space=pltpu.VMEM),
        scratch_shapes=[
            pltpu.VMEM((rblk, ncols), jnp.float32),
            pltpu.VMEM((2, rows, K), jnp.float32),
            pltpu.SemaphoreType.DMA,
            pltpu.SemaphoreType.DMA((2, NXY)),
            pltpu.SemaphoreType.DMA((2, NXY)),
        ],
        compiler_params=pltpu.CompilerParams(
            collective_id=None if _ABLATE in ("local", "minimal") else 0
        ),
    )(x)


# device time: 11554 ns/iter; 4.3782x vs baseline; 1.2909x over previous
import os

import jax
import jax.numpy as jnp
from jax import lax
from jax.experimental import pallas as pl
from jax.experimental.pallas import tpu as pltpu

K = 16
NXY = 4
_ABLATE = os.environ.get("KERNEL_ABLATE", "")


def _topk_desc(v, k):
    outs = []
    for _ in range(k):
        m = jnp.max(v, axis=1, keepdims=True)
        outs.append(m)
        v = jnp.where(v == m, -jnp.inf, v)
    return jnp.concatenate(outs, axis=1)


def _local_topk(v, k):
    _, n = v.shape
    w = 128
    m1 = v[:, :w]
    m2 = jnp.full_like(m1, -jnp.inf)
    for c in range(1, n // w):
        xc = v[:, c * w:(c + 1) * w]
        m2 = jnp.maximum(m2, jnp.minimum(m1, xc))
        m1 = jnp.maximum(m1, xc)
    return _topk_desc(jnp.concatenate([m1, m2], axis=1), k)


def kernel(x):
    rows, ncols = x.shape
    rblk = rows // NXY

    def body(xblk, out_ref, cand, send_sems, recv_sems):
        mx = lax.axis_index("x")
        my = lax.axis_index("y")
        mz = lax.axis_index("z")
        rb = 2 * mx + my
        r0 = rb * rblk

        if _ABLATE not in ("local", "minimal"):
            barrier = pltpu.get_barrier_semaphore()
            for tx in (0, 1):
                for ty in (0, 1):
                    for tz in (0, 1):
                        pl.semaphore_signal(
                            barrier, inc=1, device_id=(tx, ty, tz),
                            device_id_type=pl.DeviceIdType.MESH,
                        )

        if _ABLATE == "minimal":
            out_ref[pl.ds(r0, rblk), :] = xblk[:, :K]
            return
        if _ABLATE == "noextract":
            tk = xblk[:, :K]
        elif _ABLATE == "nostage2":
            tk = jnp.maximum(xblk[:, :K], xblk[:, K:2 * K])
            for c in range(2, 32):
                tk = jnp.maximum(tk, xblk[:, c * K:(c + 1) * K])
        else:
            tk = _local_topk(xblk[:, :], K)
        cand[mz, pl.ds(r0, rblk), :] = tk

        def for_each_peer(fn):
            for tx in (0, 1):
                for ty in (0, 1):
                    for tz in (0, 1):
                        not_self = jnp.logical_not(
                            (tx == mx) & (ty == my) & (tz == mz)
                        )
                        fn((tx, ty, tz), (tz, 2 * tx + ty), not_self)

        def peer_rdma(target, sem_idx):
            return pltpu.make_async_remote_copy(
                src_ref=cand.at[mz, pl.ds(r0, rblk), :],
                dst_ref=cand.at[sem_idx[0], pl.ds(sem_idx[1] * rblk, rblk), :],
                send_sem=send_sems.at[sem_idx],
                recv_sem=recv_sems.at[sem_idx],
                device_id=target,
                device_id_type=pl.DeviceIdType.MESH,
            )

        def start_send(target, tslot, not_self):
            @pl.when(not_self)
            def _():
                pltpu.make_async_remote_copy(
                    src_ref=cand.at[mz, pl.ds(r0, rblk), :],
                    dst_ref=cand.at[mz, pl.ds(r0, rblk), :],
                    send_sem=send_sems.at[tslot],
                    recv_sem=recv_sems.at[mz, rb],
                    device_id=target,
                    device_id_type=pl.DeviceIdType.MESH,
                ).start()

        if _ABLATE not in ("nobcast", "local"):
            pl.semaphore_wait(barrier, 8)
            for_each_peer(start_send)

            def wait_recv(target, sslot, not_self):
                @pl.when(not_self)
                def _():
                    peer_rdma(target, sslot).wait_recv()

            for_each_peer(wait_recv)

        if _ABLATE == "nomerge":
            out_ref[:, :] = cand[0]
        else:
            merged = jnp.concatenate([cand[0], cand[1]], axis=1)
            out_ref[:, :] = _topk_desc(merged, K)

        if _ABLATE not in ("nobcast", "local"):
            def wait_send(target, tslot, not_self):
                @pl.when(not_self)
                def _():
                    peer_rdma(target, tslot).wait_send()

            for_each_peer(wait_send)

    mx = lax.axis_index("x")
    my = lax.axis_index("y")
    r0 = (2 * mx + my) * rblk
    xblk = lax.dynamic_slice(x, (r0, 0), (rblk, ncols))

    return pl.pallas_call(
        body,
        out_shape=jax.ShapeDtypeStruct((rows, K), jnp.float32),
        in_specs=[pl.BlockSpec(memory_space=pltpu.VMEM)],
        out_specs=pl.BlockSpec(memory_space=pltpu.VMEM),
        scratch_shapes=[
            pltpu.VMEM((2, rows, K), jnp.float32),
            pltpu.SemaphoreType.DMA((2, NXY)),
            pltpu.SemaphoreType.DMA((2, NXY)),
        ],
        compiler_params=pltpu.CompilerParams(
            collective_id=None if _ABLATE in ("local", "minimal") else 0
        ),
    )(xblk)
